# baseline (speedup 1.0000x reference)
import jax
import jax.numpy as jnp
from jax.experimental import pallas as pl


def _dot(a, b):
    return jnp.dot(a, b, preferred_element_type=jnp.float32)


def _fused_kernel(x_ref, mm_ref,
                  w11, b11, w12, b12, w13, b13, w14, b14,
                  w21, b21, w22, b22, w23, b23, w24, b24,
                  b31, w32, b32, w33, b33, w34, b34,
                  w41, b41, w42, b42, w43, b43, w44, b44,
                  out_ref):
    xc = x_ref[0]
    init = xc

    h3 = jax.nn.relu(b31[...][None, :])
    h3 = jax.nn.relu(_dot(h3, w32[...]) + b32[...])
    h3 = jax.nn.relu(_dot(h3, w33[...]) + b33[...])
    a3 = jax.nn.relu(_dot(h3, w34[...]) + b34[...])

    c_const = (_dot(a3, w41[16:24, :]) + _dot(init, w41[40:48, :])
               + b41[...][None, :])

    mm = mm_ref[0]
    mmc = mm[:, None]
    mmr = mm[None, :]
    mm2c = jnp.concatenate([mmc - 1, mmc + 1], axis=0)
    gather = (mmr == mm2c).astype(jnp.float32)
    w41_aax = jnp.concatenate([w41[0:16, :], w41[32:40, :]], axis=0)

    for _ in range(3):
        in12 = _dot(gather, xc)
        in1 = in12[0:256]
        in2 = in12[256:512]

        h1 = jax.nn.relu(_dot(in1, w11[...]) + b11[...])
        h2 = jax.nn.relu(_dot(in2, w21[...]) + b21[...])
        h1 = jax.nn.relu(_dot(h1, w12[...]) + b12[...])
        h2 = jax.nn.relu(_dot(h2, w22[...]) + b22[...])
        h1 = jax.nn.relu(_dot(h1, w13[...]) + b13[...])
        h2 = jax.nn.relu(_dot(h2, w23[...]) + b23[...])
        a1 = jax.nn.relu(_dot(h1, w14[...]) + b14[...])
        a2 = jax.nn.relu(_dot(h2, w24[...]) + b24[...])

        a4 = jax.nn.relu(jnp.sum(xc, axis=0, keepdims=True))

        aax = jnp.concatenate([a1, a2, xc], axis=1)
        h = (_dot(aax, w41_aax)
             + _dot(a4, w41[24:32, :]) + c_const)
        h = jax.nn.relu(h)
        h = jax.nn.relu(_dot(h, w42[...]) + b42[...])
        h = jax.nn.relu(_dot(h, w43[...]) + b43[...])
        xc = _dot(h, w44[...]) + b44[...]

    out_ref[0] = xc


def kernel(x, MM, PM, params):
    J, N, d = x.shape
    (f1w1, f1b1), (f1w2, f1b2), (f1w3, f1b3), (f1w4, f1b4) = params["f1"]
    (f2w1, f2b1), (f2w2, f2b2), (f2w3, f2b3), (f2w4, f2b4) = params["f2"]
    (_unused_w31, f3b1), (f3w2, f3b2), (f3w3, f3b3), (f3w4, f3b4) = params["f3"]
    (f4w1, f4b1), (f4w2, f4b2), (f4w3, f4b3), (f4w4, f4b4) = params["f4"]
    flat = [f1w1, f1b1, f1w2, f1b2, f1w3, f1b3, f1w4, f1b4,
            f2w1, f2b1, f2w2, f2b2, f2w3, f2b3, f2w4, f2b4,
            f3b1, f3w2, f3b2, f3w3, f3b3, f3w4, f3b4,
            f4w1, f4b1, f4w2, f4b2, f4w3, f4b3, f4w4, f4b4]
    out = pl.pallas_call(
        _fused_kernel,
        out_shape=jax.ShapeDtypeStruct((J, N, d), jnp.float32),
    )(x, MM, *flat)
    return out
